# baseline (speedup 1.0000x reference)
import jax
import jax.numpy as jnp
from jax.experimental import pallas as pl
from jax.experimental.pallas import tpu as pltpu

N = 4096
R0 = 512
NIN = N // R0
RC = 1024
NC = N // RC
F32 = jnp.float32
BF16 = jnp.bfloat16
TDIMS = (((0,), (0,)), ((), ()))


def _mega(adj_ref, x_ref, w0_ref, b0_ref, w1_ref, b1_ref, wg1_ref, bg1_ref,
          wg2_ref, bg2_ref, out_ref,
          adj8v, d1s, css, d2s, v1s, v2s, logits_s, v3s, acc4t, v4s, acc5t):
    i = pl.program_id(0)

    @pl.when(i < NIN)
    def _():
        sl = pl.ds(i * R0, R0)
        blk = adj_ref[...]
        adj8v[sl, :] = blk.astype(jnp.int8)
        rs = jnp.sum(blk, axis=1, keepdims=True)
        d1 = (rs + 1.0) ** -0.5
        d1s[sl, :] = d1

        @pl.when(i == 0)
        def _():
            css[...] = jnp.zeros_like(css)

        css[...] += jnp.sum(blk, axis=0, keepdims=True)
        xw = jnp.dot(x_ref[...], w0_ref[...], preferred_element_type=F32)
        v1s[sl, :] = (d1 * xw).astype(BF16)

    @pl.when(jnp.logical_and(i >= NIN, i < NIN + NC))
    def _():
        k = i - NIN
        sl = pl.ds(k * RC, RC)

        @pl.when(k == 0)
        def _():
            d2s[...] = (css[...].T + 1.0) ** -0.5

        blk = adj8v[sl, :].astype(BF16)
        acc = jnp.dot(blk, v1s[...], preferred_element_type=F32)
        pre = acc + v1s[sl, :].astype(F32)
        d1 = d1s[sl, :]
        h1 = jax.nn.relu(d1 * pre + b0_ref[...])
        v2s[sl, :] = d1 * jnp.dot(h1, w1_ref[...], preferred_element_type=F32)

    @pl.when(jnp.logical_and(i >= NIN + NC, i < NIN + 2 * NC))
    def _():
        k = i - (NIN + NC)
        sl = pl.ds(k * RC, RC)
        blk = adj8v[sl, :].astype(BF16)
        acc = jnp.dot(blk, v2s[...].astype(BF16), preferred_element_type=F32)
        logits = d1s[sl, :] * (acc + v2s[sl, :]) + b1_ref[...]
        logits_s[sl, :] = logits
        v3 = d2s[sl, :] * jnp.dot(logits, wg1_ref[...],
                                  preferred_element_type=F32)
        v3s[sl, :] = v3

        @pl.when(k == 0)
        def _():
            acc4t[...] = jnp.zeros_like(acc4t)

        acc4t[...] += jax.lax.dot_general(v3.astype(BF16), blk, TDIMS,
                                          preferred_element_type=F32)

    @pl.when(i >= NIN + 2 * NC)
    def _():
        k = i - (NIN + 2 * NC)
        sl = pl.ds(k * RC, RC)
        blk = adj8v[sl, :].astype(BF16)
        acc4b = acc4t[:, sl].T
        t = jax.nn.relu(d2s[sl, :] * (acc4b + v3s[sl, :]) + bg1_ref[...])
        v4 = d2s[sl, :] * jnp.dot(t, wg2_ref[...], preferred_element_type=F32)
        v4s[sl, :] = v4

        @pl.when(k == 0)
        def _():
            acc5t[...] = jnp.zeros_like(acc5t)

        acc5t[...] += jax.lax.dot_general(v4.astype(BF16), blk, TDIMS,
                                          preferred_element_type=F32)

        @pl.when(k == NC - 1)
        def _():
            t2 = d2s[...] * (acc5t[...].T + v4s[...]) + bg2_ref[...]
            t3 = jnp.log(jnp.exp(t2) + 1.1)
            o = logits_s[...] * t3
            m = jnp.max(o, axis=1, keepdims=True)
            lse = m + jnp.log(jnp.sum(jnp.exp(o - m), axis=1, keepdims=True))
            out_ref[...] = o - lse


@jax.jit
def kernel(x, adj, W0, b0, W1, b1, Wg1, bg1, Wg2, bg2):
    D = x.shape[1]
    H = W0.shape[1]
    C = W1.shape[1]

    def strip(f):
        return pl.BlockSpec((R0, f),
                            lambda i: (jnp.minimum(i, NIN - 1), 0))

    def full(n, f):
        return pl.BlockSpec((n, f), lambda i: (0, 0))

    out = pl.pallas_call(
        _mega,
        grid=(NIN + 3 * NC,),
        in_specs=[strip(N), strip(D), full(D, H), full(1, H), full(H, C),
                  full(1, C), full(C, C), full(1, C), full(C, C),
                  full(1, C)],
        out_specs=full(N, C),
        out_shape=jax.ShapeDtypeStruct((N, C), F32),
        scratch_shapes=[
            pltpu.VMEM((N, N), jnp.int8),
            pltpu.VMEM((N, 1), F32),
            pltpu.VMEM((1, N), F32),
            pltpu.VMEM((N, 1), F32),
            pltpu.VMEM((N, H), BF16),
            pltpu.VMEM((N, C), F32),
            pltpu.VMEM((N, C), F32),
            pltpu.VMEM((N, C), F32),
            pltpu.VMEM((C, N), F32),
            pltpu.VMEM((N, C), F32),
            pltpu.VMEM((C, N), F32),
        ],
        compiler_params=pltpu.CompilerParams(
            vmem_limit_bytes=100 * 1024 * 1024),
    )(adj, x, W0, b0[None, :], W1, b1[None, :], Wg1, bg1[None, :], Wg2,
      bg2[None, :])

    return out


# speedup vs baseline: 1.0222x; 1.0222x over previous
import jax
import jax.numpy as jnp
from jax.experimental import pallas as pl
from jax.experimental.pallas import tpu as pltpu

N = 4096
R0 = 512
NIN = N // R0
RC = 2048
NC = N // RC
F32 = jnp.float32
BF16 = jnp.bfloat16
TDIMS = (((0,), (0,)), ((), ()))


def _mega(adj_ref, x_ref, w0_ref, b0_ref, w1_ref, b1_ref, wg1_ref, bg1_ref,
          wg2_ref, bg2_ref, out_ref,
          adj8v, d1s, css, d2s, v1s, v2s, logits_s, v3s, acc4t, v4s, acc5t):
    i = pl.program_id(0)

    @pl.when(i < NIN)
    def _():
        sl = pl.ds(i * R0, R0)
        blk = adj_ref[...]
        adj8v[sl, :] = blk.astype(jnp.int8)
        rs = jnp.sum(blk, axis=1, keepdims=True)
        d1 = (rs + 1.0) ** -0.5
        d1s[sl, :] = d1

        @pl.when(i == 0)
        def _():
            css[...] = jnp.zeros_like(css)

        css[...] += jnp.sum(blk, axis=0, keepdims=True)
        xw = jnp.dot(x_ref[...], w0_ref[...], preferred_element_type=F32)
        v1s[sl, :] = (d1 * xw).astype(BF16)

    @pl.when(jnp.logical_and(i >= NIN, i < NIN + NC))
    def _():
        k = i - NIN
        sl = pl.ds(k * RC, RC)

        @pl.when(k == 0)
        def _():
            d2s[...] = (css[...].T + 1.0) ** -0.5

        blk = adj8v[sl, :].astype(BF16)
        acc = jnp.dot(blk, v1s[...], preferred_element_type=F32)
        pre = acc + v1s[sl, :].astype(F32)
        d1 = d1s[sl, :]
        h1 = jax.nn.relu(d1 * pre + b0_ref[...])
        v2s[sl, :] = d1 * jnp.dot(h1, w1_ref[...], preferred_element_type=F32)

    @pl.when(jnp.logical_and(i >= NIN + NC, i < NIN + 2 * NC))
    def _():
        k = i - (NIN + NC)
        sl = pl.ds(k * RC, RC)
        blk = adj8v[sl, :].astype(BF16)
        acc = jnp.dot(blk, v2s[...].astype(BF16), preferred_element_type=F32)
        logits = d1s[sl, :] * (acc + v2s[sl, :]) + b1_ref[...]
        logits_s[sl, :] = logits
        v3 = d2s[sl, :] * jnp.dot(logits, wg1_ref[...],
                                  preferred_element_type=F32)
        v3s[sl, :] = v3

        @pl.when(k == 0)
        def _():
            acc4t[...] = jnp.zeros_like(acc4t)

        acc4t[...] += jax.lax.dot_general(v3.astype(BF16), blk, TDIMS,
                                          preferred_element_type=F32)

    @pl.when(i >= NIN + 2 * NC)
    def _():
        k = i - (NIN + 2 * NC)
        sl = pl.ds(k * RC, RC)
        blk = adj8v[sl, :].astype(BF16)
        acc4b = acc4t[:, sl].T
        t = jax.nn.relu(d2s[sl, :] * (acc4b + v3s[sl, :]) + bg1_ref[...])
        v4 = d2s[sl, :] * jnp.dot(t, wg2_ref[...], preferred_element_type=F32)
        v4s[sl, :] = v4

        @pl.when(k == 0)
        def _():
            acc5t[...] = jnp.zeros_like(acc5t)

        acc5t[...] += jax.lax.dot_general(v4.astype(BF16), blk, TDIMS,
                                          preferred_element_type=F32)

        @pl.when(k == NC - 1)
        def _():
            t2 = d2s[...] * (acc5t[...].T + v4s[...]) + bg2_ref[...]
            t3 = jnp.log(jnp.exp(t2) + 1.1)
            o = logits_s[...] * t3
            m = jnp.max(o, axis=1, keepdims=True)
            lse = m + jnp.log(jnp.sum(jnp.exp(o - m), axis=1, keepdims=True))
            out_ref[...] = o - lse


@jax.jit
def kernel(x, adj, W0, b0, W1, b1, Wg1, bg1, Wg2, bg2):
    D = x.shape[1]
    H = W0.shape[1]
    C = W1.shape[1]

    def strip(f):
        return pl.BlockSpec((R0, f),
                            lambda i: (jnp.minimum(i, NIN - 1), 0))

    def full(n, f):
        return pl.BlockSpec((n, f), lambda i: (0, 0))

    out = pl.pallas_call(
        _mega,
        grid=(NIN + 3 * NC,),
        in_specs=[strip(N), strip(D), full(D, H), full(1, H), full(H, C),
                  full(1, C), full(C, C), full(1, C), full(C, C),
                  full(1, C)],
        out_specs=full(N, C),
        out_shape=jax.ShapeDtypeStruct((N, C), F32),
        scratch_shapes=[
            pltpu.VMEM((N, N), jnp.int8),
            pltpu.VMEM((N, 1), F32),
            pltpu.VMEM((1, N), F32),
            pltpu.VMEM((N, 1), F32),
            pltpu.VMEM((N, H), BF16),
            pltpu.VMEM((N, C), F32),
            pltpu.VMEM((N, C), F32),
            pltpu.VMEM((N, C), F32),
            pltpu.VMEM((C, N), F32),
            pltpu.VMEM((N, C), F32),
            pltpu.VMEM((C, N), F32),
        ],
        compiler_params=pltpu.CompilerParams(
            vmem_limit_bytes=100 * 1024 * 1024),
    )(adj, x, W0, b0[None, :], W1, b1[None, :], Wg1, bg1[None, :], Wg2,
      bg2[None, :])

    return out
